# baseline (speedup 1.0000x reference)
import functools

import jax
import jax.numpy as jnp
from jax import lax
from jax.experimental import pallas as pl
from jax.experimental.pallas import tpu as pltpu
from jax.experimental.pallas import tpu_sc as plsc

_VOCAB = 1000000
_B = 4096
_L = 200
_D = 32
_N = _B * _L
_NC = 2
_NS = 16
_NW = _NC * _NS
_NPW = _N // _NW
_CH = 1024
_NCH = _NPW // _CH
_BPW = _B // _NW
_LN = 16

_PAD_ID = 1
_EOS_ID = 0


_LC = 8
_NLC = _L // _LC


def _sc_body(emb, lidT, ridT,
             matT_out, realT_out, eosT_out, rndT_out,
             idx_a, idx_b, rows_a, rows_b, trans,
             mreal, meos, sem_a, sem_b, isem_a, isem_b):
    w = lax.axis_index("s") * _NC + lax.axis_index("c")
    wb = pl.multiple_of(w * _BPW, 128)
    idxb = (idx_a, idx_b)
    rowsb = (rows_a, rows_b)
    sems = (sem_a, sem_b)
    isems = (isem_a, isem_b)
    lanes = lax.iota(jnp.int32, 16)
    ones = jnp.full((_LN,), 1.0, jnp.float32)
    zeros = jnp.zeros((_LN,), jnp.float32)

    def run_table(idxT_hbm, outT_hbm, with_masks):
        def idx_copy(slot, chunk, l):
            l0 = chunk * _LC
            return pltpu.make_async_copy(
                idxT_hbm.at[l0 + l, pl.ds(wb, _BPW)],
                idxb[slot].at[pl.ds(l * _BPW, _BPW)],
                isems[slot])

        def start_chunk(slot, chunk):
            for l in range(_LC):
                idx_copy(slot, chunk, l).start()
            for l in range(_LC):
                idx_copy(slot, chunk, l).wait()
            pltpu.make_async_copy(emb.at[idxb[slot]], rowsb[slot], sems[slot]).start()

        def finish_chunk(slot, chunk):
            pltpu.make_async_copy(emb.at[idxb[slot]], rowsb[slot], sems[slot]).wait()
            l0 = chunk * _LC
            if with_masks:
                for l in range(_LC):
                    for k in range(_BPW // _LN):
                        s = pl.ds(l * _BPW + k * _LN, _LN)
                        v = idxb[slot][s]
                        ms = pl.ds(k * _LN, _LN)
                        mreal[l, ms] = jnp.where(v != _PAD_ID, ones, zeros)
                        meos[l, ms] = jnp.where(v == _EOS_ID, ones, zeros)
                pltpu.sync_copy(mreal, realT_out.at[pl.ds(l0, _LC), pl.ds(wb, _BPW)])
                pltpu.sync_copy(meos, eosT_out.at[pl.ds(l0, _LC), pl.ds(wb, _BPW)])

            def kbody(k, carry):
                boff = pl.multiple_of(k * _LN, _LN)
                for l in range(_LC):
                    rvec = l * _BPW + k * _LN + lanes
                    for d in range(_D):
                        dv = jnp.full((16,), d, jnp.int32)
                        v = plsc.load_gather(rowsb[slot], [rvec, dv])
                        tr, sr = d // 8, d % 8
                        trans[l, tr, pl.ds(sr * 128 + boff, _LN)] = v
                return carry

            lax.fori_loop(0, _BPW // _LN, kbody, 0)
            pltpu.sync_copy(trans, outT_hbm.at[pl.ds(l0, _LC), :, w])

        start_chunk(0, 0)

        def outer(t, carry):
            for slot in range(2):
                g = t * 2 + slot
                start_chunk(1 - slot, g + 1)
                finish_chunk(slot, g)
            return carry

        lax.fori_loop(0, (_NLC - 1) // 2, outer, 0)
        finish_chunk((_NLC - 1) % 2, _NLC - 1)

    run_table(lidT, matT_out, True)
    run_table(ridT, rndT_out, False)


_mesh = plsc.VectorSubcoreMesh(core_axis_name="c", subcore_axis_name="s")

_sc_kernel = functools.partial(
    pl.kernel,
    mesh=_mesh,
    compiler_params=pltpu.CompilerParams(
        use_tc_tiling_on_sc=False, needs_layout_passes=False,
        disable_bounds_checks=True),
    out_type=[
        jax.ShapeDtypeStruct((_L, 4, _NW, 1024), jnp.float32),
        jax.ShapeDtypeStruct((_L, _B), jnp.float32),
        jax.ShapeDtypeStruct((_L, _B), jnp.float32),
        jax.ShapeDtypeStruct((_L, 4, _NW, 1024), jnp.float32),
    ],
    scratch_types=[
        pltpu.VMEM((_LC * _BPW,), jnp.int32),
        pltpu.VMEM((_LC * _BPW,), jnp.int32),
        pltpu.VMEM((_LC * _BPW, _D), jnp.float32),
        pltpu.VMEM((_LC * _BPW, _D), jnp.float32),
        pltpu.VMEM((_LC, 4, 1024), jnp.float32),
        pltpu.VMEM((_LC, _BPW), jnp.float32),
        pltpu.VMEM((_LC, _BPW), jnp.float32),
        pltpu.SemaphoreType.DMA,
        pltpu.SemaphoreType.DMA,
        pltpu.SemaphoreType.DMA,
        pltpu.SemaphoreType.DMA,
    ],
)(_sc_body)


def _vec_body(wembT, wid_ids, vecT_out, idx_full, tiles, colbuf, sem):
    w = lax.axis_index("s") * _NC + lax.axis_index("c")
    wb = pl.multiple_of(w * _BPW, 128)
    pltpu.sync_copy(wid_ids, idx_full)
    lanes = lax.iota(jnp.int32, 16)

    def group(j, carry):
        off = pl.multiple_of(wb + j * 16, 16)
        vec16 = idx_full[pl.ds(off, 16)]
        vids = [jnp.sum(jnp.where(lanes == k, vec16, 0)) for k in range(16)]
        bases = [pl.multiple_of(v - v % 128, 128) for v in vids]
        for k in range(16):
            pltpu.make_async_copy(
                wembT.at[:, pl.ds(bases[k], 128)], tiles.at[k], sem
            ).start()
        for k in range(16):
            pltpu.make_async_copy(
                wembT.at[:, pl.ds(bases[k], 128)], tiles.at[k], sem
            ).wait()
            col = jnp.full((16,), vids[k] % 128, jnp.int32)
            iv = jnp.full((16,), j * 16 + k, jnp.int32)
            top = plsc.load_gather(tiles.at[k], [lanes, col])
            bot = plsc.load_gather(tiles.at[k], [lanes + 16, col])
            plsc.store_scatter(colbuf, [lanes, iv], top)
            plsc.store_scatter(colbuf, [lanes + 16, iv], bot)
        return carry

    lax.fori_loop(0, _BPW // 16, group, 0)
    pltpu.sync_copy(colbuf, vecT_out.at[:, pl.ds(wb, _BPW)])


_vec_kernel = functools.partial(
    pl.kernel,
    mesh=plsc.VectorSubcoreMesh(core_axis_name="c", subcore_axis_name="s"),
    compiler_params=pltpu.CompilerParams(
        use_tc_tiling_on_sc=True, needs_layout_passes=False),
    out_type=jax.ShapeDtypeStruct((_D, _B), jnp.float32),
    scratch_types=[
        pltpu.VMEM((_B,), jnp.int32),
        pltpu.VMEM((16, _D, 128), jnp.float32),
        pltpu.VMEM((_D, _BPW), jnp.float32),
        pltpu.SemaphoreType.DMA,
    ],
)(_vec_body)


@jax.jit
def kernel(embedding, word_embedding0, lookup_ids, word_lookup_ids, random_ids):
    lidT = lookup_ids.T.astype(jnp.int32)
    ridT = random_ids.T.astype(jnp.int32)
    wid = word_lookup_ids.astype(jnp.int32)
    matT, realT, eosT, rndT = _sc_kernel(embedding, lidT, ridT)
    vecT = _vec_kernel(word_embedding0.T, wid)

    def untile(x):
        x5 = x.reshape(_L, 4, _NW, 8, 128)
        return x5.transpose(2, 4, 0, 1, 3).reshape(_B, _L, _D)

    return (
        untile(matT),
        realT.T,
        eosT.T,
        vecT.T,
        untile(rndT),
    )


# speedup vs baseline: 1.3886x; 1.3886x over previous
import functools

import jax
import jax.numpy as jnp
from jax import lax
from jax.experimental import pallas as pl
from jax.experimental.pallas import tpu as pltpu
from jax.experimental.pallas import tpu_sc as plsc

_VOCAB = 1000000
_B = 4096
_L = 200
_D = 32
_N = _B * _L
_NC = 2
_NS = 16
_NW = _NC * _NS
_NPW = _N // _NW
_CH = 1024
_NCH = _NPW // _CH
_BPW = _B // _NW
_LN = 16

_PAD_ID = 1
_EOS_ID = 0


_LC = 8
_NLC = _L // _LC


def _sc_body(emb, lidT, ridT,
             matT_out, realT_out, eosT_out, rndT_out,
             idx_a, idx_b, rows_a, rows_b, trans,
             mreal, meos, sem_a, sem_b, isem_a, isem_b):
    w = lax.axis_index("s") * _NC + lax.axis_index("c")
    wb = pl.multiple_of(w * _BPW, 128)
    idxb = (idx_a, idx_b)
    rowsb = (rows_a, rows_b)
    sems = (sem_a, sem_b)
    isems = (isem_a, isem_b)
    lanes = lax.iota(jnp.int32, 16)
    ones = jnp.full((_LN,), 1.0, jnp.float32)
    zeros = jnp.zeros((_LN,), jnp.float32)

    def run_table(idxT_hbm, outT_hbm, with_masks):
        def idx_copy(slot, chunk, l):
            l0 = chunk * _LC
            return pltpu.make_async_copy(
                idxT_hbm.at[l0 + l, pl.ds(wb, _BPW)],
                idxb[slot].at[pl.ds(l * _BPW, _BPW)],
                isems[slot])

        def start_chunk(slot, chunk):
            for l in range(_LC):
                idx_copy(slot, chunk, l).start()
            for l in range(_LC):
                idx_copy(slot, chunk, l).wait()
            pltpu.make_async_copy(emb.at[idxb[slot]], rowsb[slot], sems[slot]).start()

        def finish_chunk(slot, chunk):
            pltpu.make_async_copy(emb.at[idxb[slot]], rowsb[slot], sems[slot]).wait()
            l0 = chunk * _LC
            if with_masks:
                for l in range(_LC):
                    for k in range(_BPW // _LN):
                        s = pl.ds(l * _BPW + k * _LN, _LN)
                        v = idxb[slot][s]
                        ms = pl.ds(k * _LN, _LN)
                        mreal[l, ms] = jnp.where(v != _PAD_ID, ones, zeros)
                        meos[l, ms] = jnp.where(v == _EOS_ID, ones, zeros)
                pltpu.sync_copy(mreal, realT_out.at[pl.ds(l0, _LC), pl.ds(wb, _BPW)])
                pltpu.sync_copy(meos, eosT_out.at[pl.ds(l0, _LC), pl.ds(wb, _BPW)])

            def kbody(k, carry):
                boff = pl.multiple_of(k * _LN, _LN)
                for l in range(_LC):
                    rvec = l * _BPW + k * _LN + lanes
                    for half in range(2):
                        vs = [
                            plsc.load_gather(
                                rowsb[slot],
                                [rvec, jnp.full((16,), half * 16 + dd, jnp.int32)])
                            for dd in range(16)
                        ]
                        for dd in range(16):
                            d = half * 16 + dd
                            tr, sr = d // 8, d % 8
                            trans[l, tr, pl.ds(sr * 128 + boff, _LN)] = vs[dd]
                return carry

            lax.fori_loop(0, _BPW // _LN, kbody, 0)
            pltpu.sync_copy(trans, outT_hbm.at[pl.ds(l0, _LC), :, w])

        start_chunk(0, 0)

        def outer(t, carry):
            for slot in range(2):
                g = t * 2 + slot
                start_chunk(1 - slot, g + 1)
                finish_chunk(slot, g)
            return carry

        lax.fori_loop(0, (_NLC - 1) // 2, outer, 0)
        finish_chunk((_NLC - 1) % 2, _NLC - 1)

    run_table(lidT, matT_out, True)
    run_table(ridT, rndT_out, False)


_mesh = plsc.VectorSubcoreMesh(core_axis_name="c", subcore_axis_name="s")

_sc_kernel = functools.partial(
    pl.kernel,
    mesh=_mesh,
    compiler_params=pltpu.CompilerParams(
        use_tc_tiling_on_sc=False, needs_layout_passes=False,
        disable_bounds_checks=True),
    out_type=[
        jax.ShapeDtypeStruct((_L, 4, _NW, 1024), jnp.float32),
        jax.ShapeDtypeStruct((_L, _B), jnp.float32),
        jax.ShapeDtypeStruct((_L, _B), jnp.float32),
        jax.ShapeDtypeStruct((_L, 4, _NW, 1024), jnp.float32),
    ],
    scratch_types=[
        pltpu.VMEM((_LC * _BPW,), jnp.int32),
        pltpu.VMEM((_LC * _BPW,), jnp.int32),
        pltpu.VMEM((_LC * _BPW, _D), jnp.float32),
        pltpu.VMEM((_LC * _BPW, _D), jnp.float32),
        pltpu.VMEM((_LC, 4, 1024), jnp.float32),
        pltpu.VMEM((_LC, _BPW), jnp.float32),
        pltpu.VMEM((_LC, _BPW), jnp.float32),
        pltpu.SemaphoreType.DMA,
        pltpu.SemaphoreType.DMA,
        pltpu.SemaphoreType.DMA,
        pltpu.SemaphoreType.DMA,
    ],
)(_sc_body)


def _vec_body(wembT, wid_ids, vecT_out, idx_full, tiles, colbuf, sem):
    w = lax.axis_index("s") * _NC + lax.axis_index("c")
    wb = pl.multiple_of(w * _BPW, 128)
    pltpu.sync_copy(wid_ids, idx_full)
    lanes = lax.iota(jnp.int32, 16)

    def group(j, carry):
        off = pl.multiple_of(wb + j * 16, 16)
        vec16 = idx_full[pl.ds(off, 16)]
        vids = [jnp.sum(jnp.where(lanes == k, vec16, 0)) for k in range(16)]
        bases = [pl.multiple_of(v - v % 128, 128) for v in vids]
        for k in range(16):
            pltpu.make_async_copy(
                wembT.at[:, pl.ds(bases[k], 128)], tiles.at[k], sem
            ).start()
        for k in range(16):
            pltpu.make_async_copy(
                wembT.at[:, pl.ds(bases[k], 128)], tiles.at[k], sem
            ).wait()
            col = jnp.full((16,), vids[k] % 128, jnp.int32)
            iv = jnp.full((16,), j * 16 + k, jnp.int32)
            top = plsc.load_gather(tiles.at[k], [lanes, col])
            bot = plsc.load_gather(tiles.at[k], [lanes + 16, col])
            plsc.store_scatter(colbuf, [lanes, iv], top)
            plsc.store_scatter(colbuf, [lanes + 16, iv], bot)
        return carry

    lax.fori_loop(0, _BPW // 16, group, 0)
    pltpu.sync_copy(colbuf, vecT_out.at[:, pl.ds(wb, _BPW)])


_vec_kernel = functools.partial(
    pl.kernel,
    mesh=plsc.VectorSubcoreMesh(core_axis_name="c", subcore_axis_name="s"),
    compiler_params=pltpu.CompilerParams(
        use_tc_tiling_on_sc=True, needs_layout_passes=False),
    out_type=jax.ShapeDtypeStruct((_D, _B), jnp.float32),
    scratch_types=[
        pltpu.VMEM((_B,), jnp.int32),
        pltpu.VMEM((16, _D, 128), jnp.float32),
        pltpu.VMEM((_D, _BPW), jnp.float32),
        pltpu.SemaphoreType.DMA,
    ],
)(_vec_body)


@jax.jit
def kernel(embedding, word_embedding0, lookup_ids, word_lookup_ids, random_ids):
    lidT = lookup_ids.T.astype(jnp.int32)
    ridT = random_ids.T.astype(jnp.int32)
    wid = word_lookup_ids.astype(jnp.int32)
    matT, realT, eosT, rndT = _sc_kernel(embedding, lidT, ridT)
    vecT = _vec_kernel(word_embedding0.T, wid)

    def untile(x):
        x5 = x.reshape(_L, 4, _NW, 8, 128)
        return x5.transpose(2, 4, 0, 1, 3).reshape(_B, _L, _D)

    return (
        untile(matT),
        realT.T,
        eosT.T,
        vecT.T,
        untile(rndT),
    )
